# baseline (speedup 1.0000x reference)
import functools

import jax
import jax.numpy as jnp
from jax import lax
from jax.experimental import pallas as pl
from jax.experimental.pallas import tpu as pltpu
from jax.experimental.pallas import tpu_sc as plsc

N = 10000
E = 320000
H = 128
EF = 16
DIM = 3
XP = 16
TW = H + XP

NC = 2
NS = 16
NW = NC * NS
EPW = E // NW
CH = 400
NCHUNK = EPW // CH
CH3 = 200
NCHUNK3 = EPW // CH3

TE = 2560
TN = 2000


HP = H // 2


def _k1_body(tab_h, tab_x, src, dst, hs_out, hd_out, xs_out, xd_out,
             sidx_v, didx_v, hs_v, hd_v, xs_v, xd_v, sem):
    wid = lax.axis_index("s") * NC + lax.axis_index("c")

    def chunk(j, carry):
        base = pl.multiple_of(wid * EPW + j * CH, 8)
        pltpu.sync_copy(src.at[pl.ds(base, CH)], sidx_v)
        pltpu.sync_copy(dst.at[pl.ds(base, CH)], didx_v)
        cp1 = pltpu.async_copy(tab_h.at[sidx_v], hs_v, sem)
        cp2 = pltpu.async_copy(tab_h.at[didx_v], hd_v, sem)
        cp3 = pltpu.async_copy(tab_x.at[sidx_v], xs_v, sem)
        cp4 = pltpu.async_copy(tab_x.at[didx_v], xd_v, sem)
        cp1.wait()
        cp2.wait()
        cp3.wait()
        cp4.wait()
        pltpu.sync_copy(hs_v, hs_out.at[pl.ds(base, CH)])
        pltpu.sync_copy(hd_v, hd_out.at[pl.ds(base, CH)])
        pltpu.sync_copy(xs_v, xs_out.at[pl.ds(base, CH)])
        pltpu.sync_copy(xd_v, xd_out.at[pl.ds(base, CH)])
        return carry

    lax.fori_loop(0, NCHUNK, chunk, 0)


def _silu(x):
    return x * jax.nn.sigmoid(x)


def _k2_body(hs_in, hd_in, xs_in, xd_in, a,
             we1s, we1d, we1r, we1a, be1, we2, be2, watt_rep, batt,
             wc1s, wc1d, wc1r, wc1a, bc1, wc2, bc2, wc3_rep,
             msg_h_out, msg_x_out):
    bf = jnp.bfloat16
    f32 = jnp.float32
    hs = hs_in[...]
    hd = hd_in[...]
    xs = xs_in[...]
    xd = xd_in[...]
    av = a[...].astype(bf)

    def mm(x, w):
        return jnp.dot(x, w[...], preferred_element_type=f32)

    diffs = xs - xd
    rad = jnp.sum(diffs * diffs, axis=1, keepdims=True)

    def sigmoid_bf(x):
        return jax.nn.sigmoid(x.astype(bf))

    def silu_bf(x):
        xb = x.astype(bf)
        return xb * jax.nn.sigmoid(xb)

    t = silu_bf(mm(hs, wc1s) + mm(hd, wc1d) + mm(av, wc1a)
                + rad * wc1r[...] + bc1[...])
    t = silu_bf(mm(t, wc2) + bc2[...])
    scale = mm(t, wc3_rep)
    inv = 1.0 / (jnp.sqrt(rad + 1e-05) + 1.0)
    msg_x_out[...] = scale * inv * diffs

    mh = silu_bf(mm(hs, we1s) + mm(hd, we1d) + mm(av, we1a)
                 + rad * we1r[...] + be1[...])
    mh = silu_bf(mm(mh, we2) + be2[...])
    att = sigmoid_bf(mm(mh, watt_rep) + batt[...])
    msg_h_out[...] = (att * mh).astype(f32)


def _k2_edge_mlp(hs, hd, xs, xd, a, wparams):
    (we1s, we1d, we1r, we1a, be1, we2, be2, watt_rep, batt,
     wc1s, wc1d, wc1r, wc1a, bc1, wc2, bc2, wc3_rep) = wparams
    grid = (E // TE,)
    full = lambda shape: pl.BlockSpec(shape, lambda i: (0, 0))
    return pl.pallas_call(
        _k2_body,
        grid=grid,
        in_specs=[
            pl.BlockSpec((TE, H), lambda i: (i, 0)),
            pl.BlockSpec((TE, H), lambda i: (i, 0)),
            pl.BlockSpec((TE, XP), lambda i: (i, 0)),
            pl.BlockSpec((TE, XP), lambda i: (i, 0)),
            pl.BlockSpec((TE, EF), lambda i: (i, 0)),
            full((H, H)), full((H, H)), full((1, H)), full((EF, H)),
            full((1, H)), full((H, H)), full((1, H)), full((H, H)),
            full((1, 1)),
            full((H, H)), full((H, H)), full((1, H)), full((EF, H)),
            full((1, H)), full((H, H)), full((1, H)), full((H, XP)),
        ],
        out_specs=[
            pl.BlockSpec((TE, H), lambda i: (i, 0)),
            pl.BlockSpec((TE, XP), lambda i: (i, 0)),
        ],
        out_shape=[
            jax.ShapeDtypeStruct((E, H), jnp.float32),
            jax.ShapeDtypeStruct((E, XP), jnp.float32),
        ],
    )(hs, hd, xs, xd, a, we1s, we1d, we1r, we1a, be1, we2, be2, watt_rep,
      batt, wc1s, wc1d, wc1r, wc1a, bc1, wc2, bc2, wc3_rep)


_RPT8 = 624
_LAST0 = _RPT8 * (NS - 1)
_RPTL = N - _LAST0


def _make_k3_body(chunk_sz):
    nchunk = EPW // chunk_sz

    def _k3_body(msg, dst, zeros, out, idx_v, m_v, acc, sem):
        c = lax.axis_index("c")
        s = lax.axis_index("s")
        wid = s * NC + c
        r0 = pl.multiple_of(s * _RPT8, 8)

        @pl.when(s < NS - 1)
        def _():
            pltpu.sync_copy(zeros.at[pl.ds(r0, _RPT8)], acc.at[pl.ds(r0, _RPT8)])

        @pl.when(s == NS - 1)
        def _():
            pltpu.sync_copy(zeros.at[pl.ds(_LAST0, _RPTL)],
                            acc.at[pl.ds(_LAST0, _RPTL)])

        plsc.subcore_barrier()

        def chunk(j, carry):
            base = pl.multiple_of(wid * EPW + j * chunk_sz, 8)
            pltpu.sync_copy(dst.at[pl.ds(base, chunk_sz)], idx_v)
            cp1 = pltpu.async_copy(msg.at[pl.ds(base, chunk_sz)], m_v, sem)
            cp1.wait()
            pltpu.sync_copy(m_v, acc.at[idx_v], add=True)
            return carry

        lax.fori_loop(0, nchunk, chunk, 0)
        plsc.subcore_barrier()

        @pl.when(s < NS - 1)
        def _():
            pltpu.sync_copy(acc.at[pl.ds(r0, _RPT8)],
                            out.at[c, pl.ds(r0, _RPT8)])

        @pl.when(s == NS - 1)
        def _():
            pltpu.sync_copy(acc.at[pl.ds(_LAST0, _RPTL)],
                            out.at[c, pl.ds(_LAST0, _RPTL)])

    return _k3_body


def _k4_body(h, p0, p1, coords, x0, x1, wn1h, wn1g, bn1, wn2, bn2,
             h_out, c_out):
    hb = h[...]
    hagg = p0[...] + p1[...]
    nh = _silu(hb @ wn1h[...] + hagg @ wn1g[...] + bn1[...])
    h_out[...] = hb + nh @ wn2[...] + bn2[...]
    c_out[...] = coords[...] + (x0[...] + x1[...])[:, :DIM]


def _k4_node_mlp(h, p0, p1, coords, x0, x1, wn1h, wn1g, bn1, wn2, bn2):
    grid = (N // TN,)
    full = lambda shape: pl.BlockSpec(shape, lambda i: (0, 0))
    return pl.pallas_call(
        _k4_body,
        grid=grid,
        in_specs=[
            pl.BlockSpec((TN, H), lambda i: (i, 0)),
            pl.BlockSpec((TN, H), lambda i: (i, 0)),
            pl.BlockSpec((TN, H), lambda i: (i, 0)),
            pl.BlockSpec((TN, DIM), lambda i: (i, 0)),
            pl.BlockSpec((TN, XP), lambda i: (i, 0)),
            pl.BlockSpec((TN, XP), lambda i: (i, 0)),
            full((H, H)), full((H, H)), full((1, H)), full((H, H)),
            full((1, H)),
        ],
        out_specs=[
            pl.BlockSpec((TN, H), lambda i: (i, 0)),
            pl.BlockSpec((TN, DIM), lambda i: (i, 0)),
        ],
        out_shape=[
            jax.ShapeDtypeStruct((N, H), jnp.float32),
            jax.ShapeDtypeStruct((N, DIM), jnp.float32),
        ],
    )(h, p0, p1, coords, x0, x1, wn1h, wn1g, bn1, wn2, bn2)


@functools.lru_cache(maxsize=None)
def _get_sc_kernels():
    mesh = plsc.VectorSubcoreMesh(core_axis_name="c", subcore_axis_name="s")
    tiled = pltpu.CompilerParams(use_tc_tiling_on_sc=True)
    untiled = pltpu.CompilerParams(use_tc_tiling_on_sc=False)

    k1 = pl.kernel(
        _k1_body,
        out_type=[
            jax.ShapeDtypeStruct((E, HP), jnp.int32),
            jax.ShapeDtypeStruct((E, HP), jnp.int32),
            jax.ShapeDtypeStruct((E, XP), jnp.float32),
            jax.ShapeDtypeStruct((E, XP), jnp.float32),
        ],
        mesh=mesh,
        scratch_types=[
            pltpu.VMEM((CH,), jnp.int32),
            pltpu.VMEM((CH,), jnp.int32),
            pltpu.VMEM((CH, HP), jnp.int32),
            pltpu.VMEM((CH, HP), jnp.int32),
            pltpu.VMEM((CH, XP), jnp.float32),
            pltpu.VMEM((CH, XP), jnp.float32),
            pltpu.SemaphoreType.DMA,
        ],
        compiler_params=untiled,
    )

    def scatter_kernel(width, chunk_sz, params):
        return pl.kernel(
            _make_k3_body(chunk_sz),
            out_type=jax.ShapeDtypeStruct((NC, N, width), jnp.float32),
            mesh=mesh,
            scratch_types=[
                pltpu.VMEM((chunk_sz,), jnp.int32),
                pltpu.VMEM((chunk_sz, width), jnp.float32),
                pltpu.VMEM_SHARED((N, width), jnp.float32),
                pltpu.SemaphoreType.DMA,
            ],
            compiler_params=params,
        )

    k3a = scatter_kernel(H, CH3, tiled)
    k3b = scatter_kernel(XP, 2000, untiled)
    return k1, k3a, k3b


def kernel(h, coords, a, edge_index, w_e1, b_e1, w_e2, b_e2, w_att, b_att,
           w_n1, b_n1, w_n2, b_n2, w_c1, b_c1, w_c2, b_c2, w_c3):
    coords_p = jnp.pad(coords, ((0, 0), (0, XP - DIM)))
    src = edge_index[0]
    dst = edge_index[1]

    tab_h = jax.lax.bitcast_convert_type(
        h.astype(jnp.bfloat16).reshape(N, HP, 2), jnp.int32)

    _k1, _k3a, _k3b = _get_sc_kernels()
    hs_p, hd_p, xs, xd = _k1(tab_h, coords_p, src, dst)
    hs = jax.lax.bitcast_convert_type(hs_p, jnp.bfloat16).reshape(E, H)
    hd = jax.lax.bitcast_convert_type(hd_p, jnp.bfloat16).reshape(E, H)

    bf = jnp.bfloat16
    wparams = (
        w_e1[:H].astype(bf), w_e1[H:2 * H].astype(bf),
        w_e1[2 * H:2 * H + 1], w_e1[2 * H + 1:].astype(bf),
        b_e1.reshape(1, H), w_e2.astype(bf), b_e2.reshape(1, H),
        jnp.tile(w_att, (1, H)).astype(bf), b_att.reshape(1, 1),
        w_c1[:H].astype(bf), w_c1[H:2 * H].astype(bf),
        w_c1[2 * H:2 * H + 1], w_c1[2 * H + 1:].astype(bf),
        b_c1.reshape(1, H), w_c2.astype(bf), b_c2.reshape(1, H),
        jnp.tile(w_c3, (1, XP)).astype(bf),
    )
    msg_h, msg_x = _k2_edge_mlp(hs, hd, xs, xd, a, wparams)

    zeros_h = jnp.zeros((N, H), jnp.float32)
    zeros_x = jnp.zeros((N, XP), jnp.float32)
    part_h = _k3a(msg_h, dst, zeros_h)
    part_x = _k3b(msg_x, dst, zeros_x)

    h_out, coords_out = _k4_node_mlp(
        h, part_h[0], part_h[1], coords, part_x[0], part_x[1],
        w_n1[:H], w_n1[H:], b_n1.reshape(1, H), w_n2, b_n2.reshape(1, H))
    return (h_out, coords_out)


# speedup vs baseline: 2.2266x; 2.2266x over previous
import functools

import jax
import jax.numpy as jnp
from jax import lax
from jax.experimental import pallas as pl
from jax.experimental.pallas import tpu as pltpu
from jax.experimental.pallas import tpu_sc as plsc

N = 10000
E = 320000
H = 128
EF = 16
DIM = 3
XP = 16

NC = 2
NS = 16
NW = NC * NS
EPW = E // NW
CH = 400
NCHUNK = EPW // CH
CH3 = 200

TE = 2000
TN = 2000


def _k1a_body(tab, src, dst, hx_out, sidx_v, didx_v, hs_v, hd_v, sem):
    wid = lax.axis_index("s") * NC + lax.axis_index("c")

    def chunk(j, carry):
        base = pl.multiple_of(wid * EPW + j * CH, 8)
        pltpu.sync_copy(src.at[pl.ds(base, CH)], sidx_v)
        pltpu.sync_copy(dst.at[pl.ds(base, CH)], didx_v)
        cp1 = pltpu.async_copy(tab.at[sidx_v], hs_v, sem)
        cp2 = pltpu.async_copy(tab.at[didx_v], hd_v, sem)
        cp1.wait()
        cp2.wait()
        pltpu.sync_copy(hs_v, hx_out.at[pl.ds(base, CH), pl.ds(0, H)])
        pltpu.sync_copy(hd_v, hx_out.at[pl.ds(base, CH), pl.ds(H, H)])
        return carry

    lax.fori_loop(0, NCHUNK, chunk, 0)


def _k1b_body(tab, src, dst, xs_out, xd_out, sidx_v, didx_v, xs_v, xd_v, sem):
    wid = lax.axis_index("s") * NC + lax.axis_index("c")

    def chunk(j, carry):
        base = pl.multiple_of(wid * EPW + j * CH, 8)
        pltpu.sync_copy(src.at[pl.ds(base, CH)], sidx_v)
        pltpu.sync_copy(dst.at[pl.ds(base, CH)], didx_v)
        cp1 = pltpu.async_copy(tab.at[sidx_v], xs_v, sem)
        cp2 = pltpu.async_copy(tab.at[didx_v], xd_v, sem)
        cp1.wait()
        cp2.wait()
        pltpu.sync_copy(xs_v, xs_out.at[pl.ds(base, CH)])
        pltpu.sync_copy(xd_v, xd_out.at[pl.ds(base, CH)])
        return carry

    lax.fori_loop(0, NCHUNK, chunk, 0)


def _silu(x):
    return x * jax.nn.sigmoid(x)


def _k2_body(hx_in, xs_in, xd_in, a,
             we1sd, we1r, we1a, be1, we2, be2, watt_rep, batt,
             wc1sd, wc1r, wc1a, bc1, wc2, bc2, wc3_rep,
             msg_h_out, msg_x_out):
    bf = jnp.bfloat16
    f32 = jnp.float32
    hx = hx_in[...].astype(bf)
    xs = xs_in[...]
    xd = xd_in[...]
    av = a[...].astype(bf)

    def mm(x, w):
        return jnp.dot(x, w[...], preferred_element_type=f32)

    diffs = xs - xd
    rad = jnp.sum(diffs * diffs, axis=1, keepdims=True)

    def sigmoid_bf(x):
        return jax.nn.sigmoid(x.astype(bf))

    def silu_bf(x):
        xb = x.astype(bf)
        return xb * jax.nn.sigmoid(xb)

    t = silu_bf(mm(hx, wc1sd) + mm(av, wc1a) + rad * wc1r[...] + bc1[...])
    t = silu_bf(mm(t, wc2) + bc2[...])
    scale = mm(t, wc3_rep)
    inv = 1.0 / (jnp.sqrt(rad + 1e-05) + 1.0)
    msg_x_out[...] = scale * inv * diffs

    mh = silu_bf(mm(hx, we1sd) + mm(av, we1a) + rad * we1r[...] + be1[...])
    mh = silu_bf(mm(mh, we2) + be2[...])
    att = sigmoid_bf(mm(mh, watt_rep) + batt[...])
    msg_h_out[...] = (att * mh).astype(f32)


def _k2_edge_mlp(hx, xs, xd, a, wparams):
    (we1sd, we1r, we1a, be1, we2, be2, watt_rep, batt,
     wc1sd, wc1r, wc1a, bc1, wc2, bc2, wc3_rep) = wparams
    grid = (E // TE,)
    full = lambda shape: pl.BlockSpec(shape, lambda i: (0, 0))
    return pl.pallas_call(
        _k2_body,
        grid=grid,
        in_specs=[
            pl.BlockSpec((TE, 2 * H), lambda i: (i, 0)),
            pl.BlockSpec((TE, XP), lambda i: (i, 0)),
            pl.BlockSpec((TE, XP), lambda i: (i, 0)),
            pl.BlockSpec((TE, EF), lambda i: (i, 0)),
            full((2 * H, H)), full((1, H)), full((EF, H)), full((1, H)),
            full((H, H)), full((1, H)), full((H, H)), full((1, 1)),
            full((2 * H, H)), full((1, H)), full((EF, H)), full((1, H)),
            full((H, H)), full((1, H)), full((H, XP)),
        ],
        out_specs=[
            pl.BlockSpec((TE, H), lambda i: (i, 0)),
            pl.BlockSpec((TE, XP), lambda i: (i, 0)),
        ],
        out_shape=[
            jax.ShapeDtypeStruct((E, H), jnp.float32),
            jax.ShapeDtypeStruct((E, XP), jnp.float32),
        ],
    )(hx, xs, xd, a, we1sd, we1r, we1a, be1, we2, be2, watt_rep, batt,
      wc1sd, wc1r, wc1a, bc1, wc2, bc2, wc3_rep)


_RPT8 = 624
_LAST0 = _RPT8 * (NS - 1)
_RPTL = N - _LAST0


def _make_k3_body(chunk_sz):
    nchunk = EPW // chunk_sz

    def _k3_body(msg, dst, zeros, out, idx_v, m_v, acc, sem):
        c = lax.axis_index("c")
        s = lax.axis_index("s")
        wid = s * NC + c
        r0 = pl.multiple_of(s * _RPT8, 8)

        @pl.when(s < NS - 1)
        def _():
            pltpu.sync_copy(zeros.at[pl.ds(r0, _RPT8)], acc.at[pl.ds(r0, _RPT8)])

        @pl.when(s == NS - 1)
        def _():
            pltpu.sync_copy(zeros.at[pl.ds(_LAST0, _RPTL)],
                            acc.at[pl.ds(_LAST0, _RPTL)])

        plsc.subcore_barrier()

        def chunk(j, carry):
            base = pl.multiple_of(wid * EPW + j * chunk_sz, 8)
            pltpu.sync_copy(dst.at[pl.ds(base, chunk_sz)], idx_v)
            cp1 = pltpu.async_copy(msg.at[pl.ds(base, chunk_sz)], m_v, sem)
            cp1.wait()
            pltpu.sync_copy(m_v, acc.at[idx_v], add=True)
            return carry

        lax.fori_loop(0, nchunk, chunk, 0)
        plsc.subcore_barrier()

        @pl.when(s < NS - 1)
        def _():
            pltpu.sync_copy(acc.at[pl.ds(r0, _RPT8)],
                            out.at[c, pl.ds(r0, _RPT8)])

        @pl.when(s == NS - 1)
        def _():
            pltpu.sync_copy(acc.at[pl.ds(_LAST0, _RPTL)],
                            out.at[c, pl.ds(_LAST0, _RPTL)])

    return _k3_body


def _k4_body(h, p0, p1, coords, x0, x1, wn1h, wn1g, bn1, wn2, bn2,
             h_out, c_out):
    hb = h[...]
    hagg = p0[...] + p1[...]
    nh = _silu(hb @ wn1h[...] + hagg @ wn1g[...] + bn1[...])
    h_out[...] = hb + nh @ wn2[...] + bn2[...]
    c_out[...] = coords[...] + (x0[...] + x1[...])[:, :DIM]


def _k4_node_mlp(h, p0, p1, coords, x0, x1, wn1h, wn1g, bn1, wn2, bn2):
    grid = (N // TN,)
    full = lambda shape: pl.BlockSpec(shape, lambda i: (0, 0))
    return pl.pallas_call(
        _k4_body,
        grid=grid,
        in_specs=[
            pl.BlockSpec((TN, H), lambda i: (i, 0)),
            pl.BlockSpec((TN, H), lambda i: (i, 0)),
            pl.BlockSpec((TN, H), lambda i: (i, 0)),
            pl.BlockSpec((TN, DIM), lambda i: (i, 0)),
            pl.BlockSpec((TN, XP), lambda i: (i, 0)),
            pl.BlockSpec((TN, XP), lambda i: (i, 0)),
            full((H, H)), full((H, H)), full((1, H)), full((H, H)),
            full((1, H)),
        ],
        out_specs=[
            pl.BlockSpec((TN, H), lambda i: (i, 0)),
            pl.BlockSpec((TN, DIM), lambda i: (i, 0)),
        ],
        out_shape=[
            jax.ShapeDtypeStruct((N, H), jnp.float32),
            jax.ShapeDtypeStruct((N, DIM), jnp.float32),
        ],
    )(h, p0, p1, coords, x0, x1, wn1h, wn1g, bn1, wn2, bn2)


@functools.lru_cache(maxsize=None)
def _get_sc_kernels():
    mesh = plsc.VectorSubcoreMesh(core_axis_name="c", subcore_axis_name="s")
    tiled = pltpu.CompilerParams(use_tc_tiling_on_sc=True)
    untiled = pltpu.CompilerParams(use_tc_tiling_on_sc=False)

    k1a = pl.kernel(
        _k1a_body,
        out_type=jax.ShapeDtypeStruct((E, 2 * H), jnp.float32),
        mesh=mesh,
        scratch_types=[
            pltpu.VMEM((CH,), jnp.int32),
            pltpu.VMEM((CH,), jnp.int32),
            pltpu.VMEM((CH, H), jnp.float32),
            pltpu.VMEM((CH, H), jnp.float32),
            pltpu.SemaphoreType.DMA,
        ],
        compiler_params=tiled,
    )
    k1b = pl.kernel(
        _k1b_body,
        out_type=[
            jax.ShapeDtypeStruct((E, XP), jnp.float32),
            jax.ShapeDtypeStruct((E, XP), jnp.float32),
        ],
        mesh=mesh,
        scratch_types=[
            pltpu.VMEM((CH,), jnp.int32),
            pltpu.VMEM((CH,), jnp.int32),
            pltpu.VMEM((CH, XP), jnp.float32),
            pltpu.VMEM((CH, XP), jnp.float32),
            pltpu.SemaphoreType.DMA,
        ],
        compiler_params=untiled,
    )

    def scatter_kernel(width, chunk_sz, params):
        return pl.kernel(
            _make_k3_body(chunk_sz),
            out_type=jax.ShapeDtypeStruct((NC, N, width), jnp.float32),
            mesh=mesh,
            scratch_types=[
                pltpu.VMEM((chunk_sz,), jnp.int32),
                pltpu.VMEM((chunk_sz, width), jnp.float32),
                pltpu.VMEM_SHARED((N, width), jnp.float32),
                pltpu.SemaphoreType.DMA,
            ],
            compiler_params=params,
        )

    k3a = scatter_kernel(H, CH3, tiled)
    k3b = scatter_kernel(XP, 2000, untiled)
    return k1a, k1b, k3a, k3b


def kernel(h, coords, a, edge_index, w_e1, b_e1, w_e2, b_e2, w_att, b_att,
           w_n1, b_n1, w_n2, b_n2, w_c1, b_c1, w_c2, b_c2, w_c3):
    coords_p = jnp.pad(coords, ((0, 0), (0, XP - DIM)))
    src = edge_index[0]
    dst = edge_index[1]

    _k1a, _k1b, _k3a, _k3b = _get_sc_kernels()
    hx = _k1a(h, src, dst)
    xs, xd = _k1b(coords_p, src, dst)

    bf = jnp.bfloat16
    wparams = (
        w_e1[:2 * H].astype(bf), w_e1[2 * H:2 * H + 1],
        w_e1[2 * H + 1:].astype(bf), b_e1.reshape(1, H),
        w_e2.astype(bf), b_e2.reshape(1, H),
        jnp.tile(w_att, (1, H)).astype(bf), b_att.reshape(1, 1),
        w_c1[:2 * H].astype(bf), w_c1[2 * H:2 * H + 1],
        w_c1[2 * H + 1:].astype(bf), b_c1.reshape(1, H),
        w_c2.astype(bf), b_c2.reshape(1, H),
        jnp.tile(w_c3, (1, XP)).astype(bf),
    )
    msg_h, msg_x = _k2_edge_mlp(hx, xs, xd, a, wparams)

    zeros_h = jnp.zeros((N, H), jnp.float32)
    zeros_x = jnp.zeros((N, XP), jnp.float32)
    part_h = _k3a(msg_h, dst, zeros_h)
    part_x = _k3b(msg_x, dst, zeros_x)

    h_out, coords_out = _k4_node_mlp(
        h, part_h[0], part_h[1], coords, part_x[0], part_x[1],
        w_n1[:H], w_n1[H:], b_n1.reshape(1, H), w_n2, b_n2.reshape(1, H))
    return (h_out, coords_out)


# speedup vs baseline: 2.3603x; 1.0601x over previous
import functools

import jax
import jax.numpy as jnp
from jax import lax
from jax.experimental import pallas as pl
from jax.experimental.pallas import tpu as pltpu
from jax.experimental.pallas import tpu_sc as plsc

N = 10000
E = 320000
H = 128
EF = 16
DIM = 3
XP = 16

NC = 2
NS = 16
NW = NC * NS
EPW = E // NW
CH = 400
NCHUNK = EPW // CH
CH3 = 200

TE = 4000
TN = 2000


def _k1a_body(tab, src, dst, hx_out, sidx_v, didx_v, hs_v, hd_v, sem):
    wid = lax.axis_index("s") * NC + lax.axis_index("c")

    def chunk(j, carry):
        base = pl.multiple_of(wid * EPW + j * CH, 8)
        pltpu.sync_copy(src.at[pl.ds(base, CH)], sidx_v)
        pltpu.sync_copy(dst.at[pl.ds(base, CH)], didx_v)
        cp1 = pltpu.async_copy(tab.at[sidx_v], hs_v, sem)
        cp2 = pltpu.async_copy(tab.at[didx_v], hd_v, sem)
        cp1.wait()
        cp2.wait()
        pltpu.sync_copy(hs_v, hx_out.at[pl.ds(base, CH), pl.ds(0, H)])
        pltpu.sync_copy(hd_v, hx_out.at[pl.ds(base, CH), pl.ds(H, H)])
        return carry

    lax.fori_loop(0, NCHUNK, chunk, 0)


def _k1b_body(tab, src, dst, xs_out, xd_out, sidx_v, didx_v, xs_v, xd_v, sem):
    wid = lax.axis_index("s") * NC + lax.axis_index("c")

    def chunk(j, carry):
        base = pl.multiple_of(wid * EPW + j * CH, 8)
        pltpu.sync_copy(src.at[pl.ds(base, CH)], sidx_v)
        pltpu.sync_copy(dst.at[pl.ds(base, CH)], didx_v)
        cp1 = pltpu.async_copy(tab.at[sidx_v], xs_v, sem)
        cp2 = pltpu.async_copy(tab.at[didx_v], xd_v, sem)
        cp1.wait()
        cp2.wait()
        pltpu.sync_copy(xs_v, xs_out.at[pl.ds(base, CH)])
        pltpu.sync_copy(xd_v, xd_out.at[pl.ds(base, CH)])
        return carry

    lax.fori_loop(0, NCHUNK, chunk, 0)


def _silu(x):
    return x * jax.nn.sigmoid(x)


def _k2_body(hx_in, xs_in, xd_in, a,
             we1sd, we1r, we1a, be1, we2, be2, watt_rep, batt,
             wc1sd, wc1r, wc1a, bc1, wc2, bc2, wc3_rep,
             msg_h_out, msg_x_out):
    bf = jnp.bfloat16
    f32 = jnp.float32
    hx = hx_in[...].astype(bf)
    xs = xs_in[...]
    xd = xd_in[...]
    av = a[...].astype(bf)

    def mm(x, w):
        return jnp.dot(x, w[...], preferred_element_type=f32)

    diffs = xs - xd
    rad = jnp.sum(diffs * diffs, axis=1, keepdims=True)

    def sigmoid_bf(x):
        return jax.nn.sigmoid(x.astype(bf))

    def silu_bf(x):
        xb = x.astype(bf)
        return xb * jax.nn.sigmoid(xb)

    t = silu_bf(mm(hx, wc1sd) + mm(av, wc1a) + rad * wc1r[...] + bc1[...])
    t = silu_bf(mm(t, wc2) + bc2[...])
    scale = mm(t, wc3_rep)
    inv = 1.0 / (jnp.sqrt(rad + 1e-05) + 1.0)
    msg_x_out[...] = scale * inv * diffs

    mh = silu_bf(mm(hx, we1sd) + mm(av, we1a) + rad * we1r[...] + be1[...])
    mh = silu_bf(mm(mh, we2) + be2[...])
    att = sigmoid_bf(mm(mh, watt_rep) + batt[...])
    msg_h_out[...] = (att * mh).astype(f32)


def _k2_edge_mlp(hx, xs, xd, a, wparams):
    (we1sd, we1r, we1a, be1, we2, be2, watt_rep, batt,
     wc1sd, wc1r, wc1a, bc1, wc2, bc2, wc3_rep) = wparams
    grid = (E // TE,)
    full = lambda shape: pl.BlockSpec(shape, lambda i: (0, 0))
    return pl.pallas_call(
        _k2_body,
        grid=grid,
        in_specs=[
            pl.BlockSpec((TE, 2 * H), lambda i: (i, 0)),
            pl.BlockSpec((TE, XP), lambda i: (i, 0)),
            pl.BlockSpec((TE, XP), lambda i: (i, 0)),
            pl.BlockSpec((TE, EF), lambda i: (i, 0)),
            full((2 * H, H)), full((1, H)), full((EF, H)), full((1, H)),
            full((H, H)), full((1, H)), full((H, H)), full((1, 1)),
            full((2 * H, H)), full((1, H)), full((EF, H)), full((1, H)),
            full((H, H)), full((1, H)), full((H, XP)),
        ],
        out_specs=[
            pl.BlockSpec((TE, H), lambda i: (i, 0)),
            pl.BlockSpec((TE, XP), lambda i: (i, 0)),
        ],
        out_shape=[
            jax.ShapeDtypeStruct((E, H), jnp.float32),
            jax.ShapeDtypeStruct((E, XP), jnp.float32),
        ],
    )(hx, xs, xd, a, we1sd, we1r, we1a, be1, we2, be2, watt_rep, batt,
      wc1sd, wc1r, wc1a, bc1, wc2, bc2, wc3_rep)


_RPT8 = 624
_LAST0 = _RPT8 * (NS - 1)
_RPTL = N - _LAST0


def _make_k3_body(chunk_sz):
    nchunk = EPW // chunk_sz

    def _k3_body(msg, dst, zeros, out, idx_v, m_v, acc, sem):
        c = lax.axis_index("c")
        s = lax.axis_index("s")
        wid = s * NC + c
        r0 = pl.multiple_of(s * _RPT8, 8)

        @pl.when(s < NS - 1)
        def _():
            pltpu.sync_copy(zeros.at[pl.ds(r0, _RPT8)], acc.at[pl.ds(r0, _RPT8)])

        @pl.when(s == NS - 1)
        def _():
            pltpu.sync_copy(zeros.at[pl.ds(_LAST0, _RPTL)],
                            acc.at[pl.ds(_LAST0, _RPTL)])

        plsc.subcore_barrier()

        def chunk(j, carry):
            base = pl.multiple_of(wid * EPW + j * chunk_sz, 8)
            pltpu.sync_copy(dst.at[pl.ds(base, chunk_sz)], idx_v)
            cp1 = pltpu.async_copy(msg.at[pl.ds(base, chunk_sz)], m_v, sem)
            cp1.wait()
            pltpu.sync_copy(m_v, acc.at[idx_v], add=True)
            return carry

        lax.fori_loop(0, nchunk, chunk, 0)
        plsc.subcore_barrier()

        @pl.when(s < NS - 1)
        def _():
            pltpu.sync_copy(acc.at[pl.ds(r0, _RPT8)],
                            out.at[c, pl.ds(r0, _RPT8)])

        @pl.when(s == NS - 1)
        def _():
            pltpu.sync_copy(acc.at[pl.ds(_LAST0, _RPTL)],
                            out.at[c, pl.ds(_LAST0, _RPTL)])

    return _k3_body


def _k4_body(h, p0, p1, coords, x0, x1, wn1h, wn1g, bn1, wn2, bn2,
             h_out, c_out):
    hb = h[...]
    hagg = p0[...] + p1[...]
    nh = _silu(hb @ wn1h[...] + hagg @ wn1g[...] + bn1[...])
    h_out[...] = hb + nh @ wn2[...] + bn2[...]
    c_out[...] = coords[...] + (x0[...] + x1[...])[:, :DIM]


def _k4_node_mlp(h, p0, p1, coords, x0, x1, wn1h, wn1g, bn1, wn2, bn2):
    grid = (N // TN,)
    full = lambda shape: pl.BlockSpec(shape, lambda i: (0, 0))
    return pl.pallas_call(
        _k4_body,
        grid=grid,
        in_specs=[
            pl.BlockSpec((TN, H), lambda i: (i, 0)),
            pl.BlockSpec((TN, H), lambda i: (i, 0)),
            pl.BlockSpec((TN, H), lambda i: (i, 0)),
            pl.BlockSpec((TN, DIM), lambda i: (i, 0)),
            pl.BlockSpec((TN, XP), lambda i: (i, 0)),
            pl.BlockSpec((TN, XP), lambda i: (i, 0)),
            full((H, H)), full((H, H)), full((1, H)), full((H, H)),
            full((1, H)),
        ],
        out_specs=[
            pl.BlockSpec((TN, H), lambda i: (i, 0)),
            pl.BlockSpec((TN, DIM), lambda i: (i, 0)),
        ],
        out_shape=[
            jax.ShapeDtypeStruct((N, H), jnp.float32),
            jax.ShapeDtypeStruct((N, DIM), jnp.float32),
        ],
    )(h, p0, p1, coords, x0, x1, wn1h, wn1g, bn1, wn2, bn2)


@functools.lru_cache(maxsize=None)
def _get_sc_kernels():
    mesh = plsc.VectorSubcoreMesh(core_axis_name="c", subcore_axis_name="s")
    tiled = pltpu.CompilerParams(use_tc_tiling_on_sc=True)
    untiled = pltpu.CompilerParams(use_tc_tiling_on_sc=False)

    k1a = pl.kernel(
        _k1a_body,
        out_type=jax.ShapeDtypeStruct((E, 2 * H), jnp.float32),
        mesh=mesh,
        scratch_types=[
            pltpu.VMEM((CH,), jnp.int32),
            pltpu.VMEM((CH,), jnp.int32),
            pltpu.VMEM((CH, H), jnp.float32),
            pltpu.VMEM((CH, H), jnp.float32),
            pltpu.SemaphoreType.DMA,
        ],
        compiler_params=tiled,
    )
    k1b = pl.kernel(
        _k1b_body,
        out_type=[
            jax.ShapeDtypeStruct((E, XP), jnp.float32),
            jax.ShapeDtypeStruct((E, XP), jnp.float32),
        ],
        mesh=mesh,
        scratch_types=[
            pltpu.VMEM((CH,), jnp.int32),
            pltpu.VMEM((CH,), jnp.int32),
            pltpu.VMEM((CH, XP), jnp.float32),
            pltpu.VMEM((CH, XP), jnp.float32),
            pltpu.SemaphoreType.DMA,
        ],
        compiler_params=untiled,
    )

    def scatter_kernel(width, chunk_sz, params):
        return pl.kernel(
            _make_k3_body(chunk_sz),
            out_type=jax.ShapeDtypeStruct((NC, N, width), jnp.float32),
            mesh=mesh,
            scratch_types=[
                pltpu.VMEM((chunk_sz,), jnp.int32),
                pltpu.VMEM((chunk_sz, width), jnp.float32),
                pltpu.VMEM_SHARED((N, width), jnp.float32),
                pltpu.SemaphoreType.DMA,
            ],
            compiler_params=params,
        )

    k3a = scatter_kernel(H, CH3, tiled)
    k3b = scatter_kernel(XP, 2000, untiled)
    return k1a, k1b, k3a, k3b


def kernel(h, coords, a, edge_index, w_e1, b_e1, w_e2, b_e2, w_att, b_att,
           w_n1, b_n1, w_n2, b_n2, w_c1, b_c1, w_c2, b_c2, w_c3):
    coords_p = jnp.pad(coords, ((0, 0), (0, XP - DIM)))
    src = edge_index[0]
    dst = edge_index[1]

    _k1a, _k1b, _k3a, _k3b = _get_sc_kernels()
    hx = _k1a(h, src, dst)
    xs, xd = _k1b(coords_p, src, dst)

    bf = jnp.bfloat16
    wparams = (
        w_e1[:2 * H].astype(bf), w_e1[2 * H:2 * H + 1],
        w_e1[2 * H + 1:].astype(bf), b_e1.reshape(1, H),
        w_e2.astype(bf), b_e2.reshape(1, H),
        jnp.tile(w_att, (1, H)).astype(bf), b_att.reshape(1, 1),
        w_c1[:2 * H].astype(bf), w_c1[2 * H:2 * H + 1],
        w_c1[2 * H + 1:].astype(bf), b_c1.reshape(1, H),
        w_c2.astype(bf), b_c2.reshape(1, H),
        jnp.tile(w_c3, (1, XP)).astype(bf),
    )
    msg_h, msg_x = _k2_edge_mlp(hx, xs, xd, a, wparams)

    zeros_h = jnp.zeros((N, H), jnp.float32)
    zeros_x = jnp.zeros((N, XP), jnp.float32)
    part_h = _k3a(msg_h, dst, zeros_h)
    part_x = _k3b(msg_x, dst, zeros_x)

    h_out, coords_out = _k4_node_mlp(
        h, part_h[0], part_h[1], coords, part_x[0], part_x[1],
        w_n1[:H], w_n1[H:], b_n1.reshape(1, H), w_n2, b_n2.reshape(1, H))
    return (h_out, coords_out)
